# baseline (speedup 1.0000x reference)
import jax
import jax.numpy as jnp
from jax import lax
from jax.experimental import pallas as pl
from jax.experimental.pallas import tpu as pltpu
from jax.experimental.pallas import tpu_sc as plsc

_NC = 2
_NS = 16
_NW = _NC * _NS


_HB = 256


def _router_gather(router_hbm, x_hbm, out_hbm, table_sh, idx_v, out_v, isem, hsem, gsem):
    c = lax.axis_index("c")
    s = lax.axis_index("s")
    wid = s * _NC + c
    bpw = idx_v.shape[0]
    wpr = x_hbm.shape[1] // bpw
    row = wid // wpr
    col = (wid % wpr) * bpw
    idx_cp = pltpu.async_copy(x_hbm.at[row, pl.ds(col, bpw)], idx_v, isem)

    @pl.when(s == 0)
    def _():
        pltpu.sync_copy(router_hbm, table_sh)

    idx_cp.wait()
    gh = pltpu.async_copy(
        router_hbm.at[idx_v.at[pl.ds(0, _HB)]], out_v.at[pl.ds(0, _HB)], hsem
    )
    plsc.subcore_barrier()
    gs = pltpu.async_copy(
        table_sh.at[idx_v.at[pl.ds(_HB, bpw - _HB)]],
        out_v.at[pl.ds(_HB, bpw - _HB)],
        gsem,
    )
    gh.wait()
    gs.wait()
    pltpu.sync_copy(out_v, out_hbm.at[row, pl.ds(col, bpw)])


def kernel(x, router):
    bpw = x.size // _NW
    mesh = plsc.VectorSubcoreMesh(core_axis_name="c", subcore_axis_name="s")
    return pl.kernel(
        _router_gather,
        out_type=jax.ShapeDtypeStruct(x.shape, jnp.int32),
        mesh=mesh,
        compiler_params=pltpu.CompilerParams(needs_layout_passes=False),
        scratch_types=[
            pltpu.VMEM_SHARED((router.shape[0],), jnp.int32),
            pltpu.VMEM((bpw,), jnp.int32),
            pltpu.VMEM((bpw,), jnp.int32),
            pltpu.SemaphoreType.DMA,
            pltpu.SemaphoreType.DMA,
            pltpu.SemaphoreType.DMA,
        ],
    )(router, x)


# speedup vs baseline: 1.0269x; 1.0269x over previous
import jax
import jax.numpy as jnp
from jax import lax
from jax.experimental import pallas as pl
from jax.experimental.pallas import tpu as pltpu
from jax.experimental.pallas import tpu_sc as plsc

_NC = 2
_NS = 16
_NW = _NC * _NS


def _router_gather(router_hbm, x_hbm, out_hbm, table_sh, idx_v, out_v, isem, gsem):
    c = lax.axis_index("c")
    s = lax.axis_index("s")
    wid = s * _NC + c
    bpw = idx_v.shape[0]
    wpr = x_hbm.shape[1] // bpw
    row = wid // wpr
    col = (wid % wpr) * bpw
    idx_cp = pltpu.async_copy(x_hbm.at[row, pl.ds(col, bpw)], idx_v, isem)

    @pl.when(s == 0)
    def _():
        pltpu.sync_copy(router_hbm, table_sh)

    plsc.subcore_barrier()
    idx_cp.wait()
    pltpu.async_copy(table_sh.at[idx_v], out_v, gsem).wait()
    pltpu.sync_copy(out_v, out_hbm.at[row, pl.ds(col, bpw)])


def kernel(x, router):
    bpw = x.size // _NW
    mesh = plsc.VectorSubcoreMesh(core_axis_name="c", subcore_axis_name="s")
    return pl.kernel(
        _router_gather,
        out_type=jax.ShapeDtypeStruct(x.shape, jnp.int32),
        mesh=mesh,
        compiler_params=pltpu.CompilerParams(needs_layout_passes=False),
        scratch_types=[
            pltpu.VMEM_SHARED((router.shape[0],), jnp.int32),
            pltpu.VMEM((bpw,), jnp.int32),
            pltpu.VMEM((bpw,), jnp.int32),
            pltpu.SemaphoreType.DMA,
            pltpu.SemaphoreType.DMA,
        ],
    )(router, x)
